# baseline (speedup 1.0000x reference)
import jax
import jax.numpy as jnp
from jax.experimental import pallas as pl
from jax.experimental.pallas import tpu as pltpu

B = 1024
IN = 256
CH = 512
E = 24768
N = 512
DI = 128
DH = 128
DY = 64
EC = 1376
NE = E // EC
SG = 32


def _enc_body(x_ref, wf1_ref, bf1_ref, wx1_ref, bx1_ref, wx2_ref, bx2_ref,
              h_ref, ex_ref):
    x = x_ref[...]
    h_ref[...] = jnp.maximum(
        jnp.dot(x, wf1_ref[...], preferred_element_type=jnp.float32) + bf1_ref[...], 0.0)
    hx = jnp.maximum(
        jnp.dot(x, wx1_ref[...], preferred_element_type=jnp.float32) + bx1_ref[...], 0.0)
    ex_ref[...] = jnp.dot(hx, wx2_ref[...], preferred_element_type=jnp.float32) + bx2_ref[...]


def _dist_body(h_ref, wf2_ref, bf2_ref, emb_ref, bd1_ref, bd2_ref,
               ind_ref, diff_ref, perp_ref, b1g_ref, b2g_ref,
               dot_acc, fsq_acc, esq_acc, fcA, fcB):
    e = pl.program_id(0)
    even = e % 2 == 0

    @pl.when(jnp.logical_and(e < NE, even))
    def _():
        fcA[...] = (jnp.dot(h_ref[...], wf2_ref[:, 0, 0, :],
                            preferred_element_type=jnp.float32) + bf2_ref[0])

    @pl.when(jnp.logical_and(e < NE, jnp.logical_not(even)))
    def _():
        fcB[...] = (jnp.dot(h_ref[...], wf2_ref[:, 0, 0, :],
                            preferred_element_type=jnp.float32) + bf2_ref[0])

    def _consume(fc_ref):
        fc = fc_ref[...]
        pdot = jnp.dot(fc, emb_ref[...], preferred_element_type=jnp.float32)
        pfsq = jnp.sum((fc * fc).reshape(B // 8, 8, EC), axis=0)
        ec2 = emb_ref[...]
        pesq = jnp.sum(ec2 * ec2, axis=0, keepdims=True)

        @pl.when(e == 1)
        def _():
            dot_acc[...] = pdot
            fsq_acc[...] = pfsq
            esq_acc[...] = pesq

        @pl.when(e > 1)
        def _():
            dot_acc[...] += pdot
            fsq_acc[...] += pfsq
            esq_acc[...] += pesq

    @pl.when(jnp.logical_and(e > 0, jnp.logical_not(even)))
    def _():
        _consume(fcA)

    @pl.when(jnp.logical_and(e > 0, even))
    def _():
        _consume(fcB)

    @pl.when(e == NE)
    def _():
        disc = 2.0 * dot_acc[...] - esq_acc[...]
        maxv = jnp.max(disc, axis=1, keepdims=True)
        lane = jax.lax.broadcasted_iota(jnp.int32, (B, N), 1)
        ind = jnp.min(jnp.where(disc == maxv, lane, N), axis=1, keepdims=True)
        ind_ref[...] = ind
        fsq_tot = jnp.sum(fsq_acc[...])
        diff_ref[...] = (fsq_tot - jnp.sum(maxv)).reshape(1, 1) * (1.0 / (B * E))
        onehot = jnp.where(ind == lane, 1.0, 0.0)
        counts = jnp.sum(onehot, axis=0, keepdims=True)
        p = counts * (1.0 / B)
        perp_ref[...] = jnp.exp(-jnp.sum(p * jnp.log(p + 1e-10))).reshape(1, 1)
        b1g_ref[...] = jnp.dot(onehot, bd1_ref[...], preferred_element_type=jnp.float32)
        b2g_ref[...] = jnp.dot(onehot, bd2_ref[...], preferred_element_type=jnp.float32)


def _dec_body(ind_sm, ex_ref, b1g_ref, b2g_ref, *refs):
    w1 = refs[0:SG]
    w2 = refs[SG:2 * SG]
    out_ref = refs[2 * SG]
    ex = ex_ref[...]
    rows = []
    for k in range(SG):
        exk = ex[k:k + 1, :]
        h1 = jnp.maximum(
            jnp.dot(exk, w1[k][0], preferred_element_type=jnp.float32)
            + b1g_ref[k:k + 1, :], 0.0)
        rows.append(
            jnp.dot(h1, w2[k][0], preferred_element_type=jnp.float32)
            + b2g_ref[k:k + 1, :])
    out_ref[...] = jnp.concatenate(rows, axis=0)


def kernel(x, Wf1, bf1, Wf2, bf2, Wx1, bx1, Wx2, bx2, embed, Wd1, bd1, Wd2, bd2):
    h, ex = pl.pallas_call(
        _enc_body,
        out_shape=[jax.ShapeDtypeStruct((B, CH), jnp.float32),
                   jax.ShapeDtypeStruct((B, DI), jnp.float32)],
    )(x, Wf1, bf1.reshape(1, CH), Wx1, bx1.reshape(1, CH), Wx2, bx2.reshape(1, DI))

    ind2, diff2, perp2, b1g, b2g = pl.pallas_call(
        _dist_body,
        grid=(NE + 1,),
        in_specs=[
            pl.BlockSpec((B, CH), lambda e: (0, 0)),
            pl.BlockSpec((CH, 1, 1, EC), lambda e: (0, jnp.minimum(e, NE - 1), 0, 0)),
            pl.BlockSpec((1, 1, EC), lambda e: (jnp.minimum(e, NE - 1), 0, 0)),
            pl.BlockSpec((EC, N), lambda e: (jnp.maximum(e - 1, 0), 0)),
            pl.BlockSpec((N, DH), lambda e: (0, 0)),
            pl.BlockSpec((N, DY), lambda e: (0, 0)),
        ],
        out_specs=[
            pl.BlockSpec((B, 1), lambda e: (0, 0)),
            pl.BlockSpec((1, 1), lambda e: (0, 0)),
            pl.BlockSpec((1, 1), lambda e: (0, 0)),
            pl.BlockSpec((B, DH), lambda e: (0, 0)),
            pl.BlockSpec((B, DY), lambda e: (0, 0)),
        ],
        out_shape=[jax.ShapeDtypeStruct((B, 1), jnp.int32),
                   jax.ShapeDtypeStruct((1, 1), jnp.float32),
                   jax.ShapeDtypeStruct((1, 1), jnp.float32),
                   jax.ShapeDtypeStruct((B, DH), jnp.float32),
                   jax.ShapeDtypeStruct((B, DY), jnp.float32)],
        scratch_shapes=[pltpu.VMEM((B, N), jnp.float32),
                        pltpu.VMEM((8, EC), jnp.float32),
                        pltpu.VMEM((1, N), jnp.float32),
                        pltpu.VMEM((B, EC), jnp.float32),
                        pltpu.VMEM((B, EC), jnp.float32)],
        compiler_params=pltpu.CompilerParams(vmem_limit_bytes=63 * 1024 * 1024),
    )(h, Wf2.reshape(CH, NE, 1, EC), bf2.reshape(NE, 1, EC), embed, bd1, bd2)

    ind_flat = ind2.reshape(B)

    def _wspec(k, shape):
        return pl.BlockSpec(shape, lambda b, ind, k=k: (ind[SG * b + k],) + (0,) * (len(shape) - 1))

    dec = pl.pallas_call(
        _dec_body,
        grid_spec=pltpu.PrefetchScalarGridSpec(
            num_scalar_prefetch=1,
            grid=(B // SG,),
            in_specs=(
                [pl.BlockSpec((SG, DI), lambda b, ind: (b, 0)),
                 pl.BlockSpec((SG, DH), lambda b, ind: (b, 0)),
                 pl.BlockSpec((SG, DY), lambda b, ind: (b, 0))]
                + [_wspec(k, (1, DI, DH)) for k in range(SG)]
                + [_wspec(k, (1, DH, DY)) for k in range(SG)]
            ),
            out_specs=pl.BlockSpec((SG, DY), lambda b, ind: (b, 0)),
        ),
        out_shape=jax.ShapeDtypeStruct((B, DY), jnp.float32),
    )(ind_flat, ex, b1g, b2g, *([Wd1] * SG), *([Wd2] * SG))

    return (dec, diff2[0, 0], ind_flat, perp2[0, 0])


# speedup vs baseline: 1.0380x; 1.0380x over previous
import jax
import jax.numpy as jnp
from jax.experimental import pallas as pl
from jax.experimental.pallas import tpu as pltpu

B = 1024
IN = 256
CH = 512
E = 24768
N = 512
DI = 128
DH = 128
DY = 64
EC = 4128
NE = E // EC
SG = 32


def _enc_body(x_ref, wf1_ref, bf1_ref, wx1_ref, bx1_ref, wx2_ref, bx2_ref,
              h_ref, ex_ref):
    x = x_ref[...]
    h_ref[...] = jnp.maximum(
        jnp.dot(x, wf1_ref[...], preferred_element_type=jnp.float32) + bf1_ref[...], 0.0)
    hx = jnp.maximum(
        jnp.dot(x, wx1_ref[...], preferred_element_type=jnp.float32) + bx1_ref[...], 0.0)
    ex_ref[...] = jnp.dot(hx, wx2_ref[...], preferred_element_type=jnp.float32) + bx2_ref[...]


def _dist_body(h_ref, wf2_ref, bf2_ref, emb_ref, bd1_ref, bd2_ref,
               ind_ref, diff_ref, perp_ref, b1g_ref, b2g_ref,
               dot_acc, fsq_acc, esq_acc):
    e = pl.program_id(0)
    fc = (jnp.dot(h_ref[...], wf2_ref[:, 0, 0, :], preferred_element_type=jnp.float32)
          + bf2_ref[0])
    ec = emb_ref[...]
    pdot = jnp.dot(fc, ec, preferred_element_type=jnp.float32)
    pfsq = jnp.sum((fc * fc).reshape(B // 8, 8, EC), axis=0)
    pesq = jnp.sum(ec * ec, axis=0, keepdims=True)

    @pl.when(e == 0)
    def _():
        dot_acc[...] = pdot
        fsq_acc[...] = pfsq
        esq_acc[...] = pesq

    @pl.when(e > 0)
    def _():
        dot_acc[...] += pdot
        fsq_acc[...] += pfsq
        esq_acc[...] += pesq

    @pl.when(e == NE - 1)
    def _():
        disc = 2.0 * dot_acc[...] - esq_acc[...]
        maxv = jnp.max(disc, axis=1, keepdims=True)
        lane = jax.lax.broadcasted_iota(jnp.int32, (B, N), 1)
        ind = jnp.min(jnp.where(disc == maxv, lane, N), axis=1, keepdims=True)
        ind_ref[...] = ind
        fsq_tot = jnp.sum(fsq_acc[...])
        diff_ref[...] = (fsq_tot - jnp.sum(maxv)).reshape(1, 1) * (1.0 / (B * E))
        onehot = jnp.where(ind == lane, 1.0, 0.0)
        counts = jnp.sum(onehot, axis=0, keepdims=True)
        p = counts * (1.0 / B)
        perp_ref[...] = jnp.exp(-jnp.sum(p * jnp.log(p + 1e-10))).reshape(1, 1)
        b1g_ref[...] = jnp.dot(onehot, bd1_ref[...], preferred_element_type=jnp.float32)
        b2g_ref[...] = jnp.dot(onehot, bd2_ref[...], preferred_element_type=jnp.float32)


def _dec_body(ind_sm, ex_ref, b1g_ref, b2g_ref, *refs):
    w1 = refs[0:SG]
    w2 = refs[SG:2 * SG]
    out_ref = refs[2 * SG]
    ex = ex_ref[...]
    rows = []
    for k in range(SG):
        exk = ex[k:k + 1, :]
        h1 = jnp.maximum(
            jnp.dot(exk, w1[k][0], preferred_element_type=jnp.float32)
            + b1g_ref[k:k + 1, :], 0.0)
        rows.append(
            jnp.dot(h1, w2[k][0], preferred_element_type=jnp.float32)
            + b2g_ref[k:k + 1, :])
    out_ref[...] = jnp.concatenate(rows, axis=0)


def kernel(x, Wf1, bf1, Wf2, bf2, Wx1, bx1, Wx2, bx2, embed, Wd1, bd1, Wd2, bd2):
    h, ex = pl.pallas_call(
        _enc_body,
        out_shape=[jax.ShapeDtypeStruct((B, CH), jnp.float32),
                   jax.ShapeDtypeStruct((B, DI), jnp.float32)],
    )(x, Wf1, bf1.reshape(1, CH), Wx1, bx1.reshape(1, CH), Wx2, bx2.reshape(1, DI))

    ind2, diff2, perp2, b1g, b2g = pl.pallas_call(
        _dist_body,
        grid=(NE,),
        in_specs=[
            pl.BlockSpec((B, CH), lambda e: (0, 0)),
            pl.BlockSpec((CH, 1, 1, EC), lambda e: (0, e, 0, 0)),
            pl.BlockSpec((1, 1, EC), lambda e: (e, 0, 0)),
            pl.BlockSpec((EC, N), lambda e: (e, 0)),
            pl.BlockSpec((N, DH), lambda e: (0, 0)),
            pl.BlockSpec((N, DY), lambda e: (0, 0)),
        ],
        out_specs=[
            pl.BlockSpec((B, 1), lambda e: (0, 0)),
            pl.BlockSpec((1, 1), lambda e: (0, 0)),
            pl.BlockSpec((1, 1), lambda e: (0, 0)),
            pl.BlockSpec((B, DH), lambda e: (0, 0)),
            pl.BlockSpec((B, DY), lambda e: (0, 0)),
        ],
        out_shape=[jax.ShapeDtypeStruct((B, 1), jnp.int32),
                   jax.ShapeDtypeStruct((1, 1), jnp.float32),
                   jax.ShapeDtypeStruct((1, 1), jnp.float32),
                   jax.ShapeDtypeStruct((B, DH), jnp.float32),
                   jax.ShapeDtypeStruct((B, DY), jnp.float32)],
        scratch_shapes=[pltpu.VMEM((B, N), jnp.float32),
                        pltpu.VMEM((8, EC), jnp.float32),
                        pltpu.VMEM((1, N), jnp.float32)],
        compiler_params=pltpu.CompilerParams(vmem_limit_bytes=63 * 1024 * 1024),
    )(h, Wf2.reshape(CH, NE, 1, EC), bf2.reshape(NE, 1, EC), embed, bd1, bd2)

    ind_flat = ind2.reshape(B)

    def _wspec(k, shape):
        return pl.BlockSpec(shape, lambda b, ind, k=k: (ind[SG * b + k],) + (0,) * (len(shape) - 1))

    dec = pl.pallas_call(
        _dec_body,
        grid_spec=pltpu.PrefetchScalarGridSpec(
            num_scalar_prefetch=1,
            grid=(B // SG,),
            in_specs=(
                [pl.BlockSpec((SG, DI), lambda b, ind: (b, 0)),
                 pl.BlockSpec((SG, DH), lambda b, ind: (b, 0)),
                 pl.BlockSpec((SG, DY), lambda b, ind: (b, 0))]
                + [_wspec(k, (1, DI, DH)) for k in range(SG)]
                + [_wspec(k, (1, DH, DY)) for k in range(SG)]
            ),
            out_specs=pl.BlockSpec((SG, DY), lambda b, ind: (b, 0)),
        ),
        out_shape=jax.ShapeDtypeStruct((B, DY), jnp.float32),
    )(ind_flat, ex, b1g, b2g, *([Wd1] * SG), *([Wd2] * SG))

    return (dec, diff2[0, 0], ind_flat, perp2[0, 0])


# speedup vs baseline: 1.0483x; 1.0100x over previous
import jax
import jax.numpy as jnp
from jax.experimental import pallas as pl
from jax.experimental.pallas import tpu as pltpu

B = 1024
IN = 256
CH = 512
E = 24768
N = 512
DI = 128
DH = 128
DY = 64
EC = 4128
NE = E // EC
SG = 32


def _dist_body(x_ref, wf1_ref, bf1_ref, wx1_ref, bx1_ref, wx2_ref, bx2_ref,
               wf2_ref, bf2_ref, emb_ref, bd1_ref, bd2_ref,
               ind_ref, diff_ref, perp_ref, b1g_ref, b2g_ref, ex_ref,
               h_s, dot_acc, fsq_acc, esq_acc):
    e = pl.program_id(0)

    @pl.when(e == 0)
    def _():
        x = x_ref[...]
        h_s[...] = jnp.maximum(
            jnp.dot(x, wf1_ref[...], preferred_element_type=jnp.float32)
            + bf1_ref[...], 0.0)
        hx = jnp.maximum(
            jnp.dot(x, wx1_ref[...], preferred_element_type=jnp.float32)
            + bx1_ref[...], 0.0)
        ex_ref[...] = (jnp.dot(hx, wx2_ref[...], preferred_element_type=jnp.float32)
                       + bx2_ref[...])

    fc = (jnp.dot(h_s[...], wf2_ref[:, 0, 0, :], preferred_element_type=jnp.float32)
          + bf2_ref[0])
    ec = emb_ref[...]
    pdot = jnp.dot(fc, ec, preferred_element_type=jnp.float32)
    pfsq = jnp.sum((fc * fc).reshape(B // 8, 8, EC), axis=0)
    pesq = jnp.sum(ec * ec, axis=0, keepdims=True)

    @pl.when(e == 0)
    def _():
        dot_acc[...] = pdot
        fsq_acc[...] = pfsq
        esq_acc[...] = pesq

    @pl.when(e > 0)
    def _():
        dot_acc[...] += pdot
        fsq_acc[...] += pfsq
        esq_acc[...] += pesq

    @pl.when(e == NE - 1)
    def _():
        disc = 2.0 * dot_acc[...] - esq_acc[...]
        maxv = jnp.max(disc, axis=1, keepdims=True)
        lane = jax.lax.broadcasted_iota(jnp.int32, (B, N), 1)
        ind = jnp.min(jnp.where(disc == maxv, lane, N), axis=1, keepdims=True)
        ind_ref[...] = ind
        fsq_tot = jnp.sum(fsq_acc[...])
        diff_ref[...] = (fsq_tot - jnp.sum(maxv)).reshape(1, 1) * (1.0 / (B * E))
        onehot = jnp.where(ind == lane, 1.0, 0.0)
        counts = jnp.sum(onehot, axis=0, keepdims=True)
        p = counts * (1.0 / B)
        perp_ref[...] = jnp.exp(-jnp.sum(p * jnp.log(p + 1e-10))).reshape(1, 1)
        b1g_ref[...] = jnp.dot(onehot, bd1_ref[...], preferred_element_type=jnp.float32)
        b2g_ref[...] = jnp.dot(onehot, bd2_ref[...], preferred_element_type=jnp.float32)


def _dec_body(ind_sm, ex_ref, b1g_ref, b2g_ref, *refs):
    w1 = refs[0:SG]
    w2 = refs[SG:2 * SG]
    out_ref = refs[2 * SG]
    ex = ex_ref[...]
    rows = []
    for k in range(SG):
        exk = ex[k:k + 1, :]
        h1 = jnp.maximum(
            jnp.dot(exk, w1[k][0], preferred_element_type=jnp.float32)
            + b1g_ref[k:k + 1, :], 0.0)
        rows.append(
            jnp.dot(h1, w2[k][0], preferred_element_type=jnp.float32)
            + b2g_ref[k:k + 1, :])
    out_ref[...] = jnp.concatenate(rows, axis=0)


def kernel(x, Wf1, bf1, Wf2, bf2, Wx1, bx1, Wx2, bx2, embed, Wd1, bd1, Wd2, bd2):
    ind2, diff2, perp2, b1g, b2g, ex = pl.pallas_call(
        _dist_body,
        grid=(NE,),
        in_specs=[
            pl.BlockSpec((B, IN), lambda e: (0, 0)),
            pl.BlockSpec((IN, CH), lambda e: (0, 0)),
            pl.BlockSpec((1, CH), lambda e: (0, 0)),
            pl.BlockSpec((IN, CH), lambda e: (0, 0)),
            pl.BlockSpec((1, CH), lambda e: (0, 0)),
            pl.BlockSpec((CH, DI), lambda e: (0, 0)),
            pl.BlockSpec((1, DI), lambda e: (0, 0)),
            pl.BlockSpec((CH, 1, 1, EC), lambda e: (0, e, 0, 0)),
            pl.BlockSpec((1, 1, EC), lambda e: (e, 0, 0)),
            pl.BlockSpec((EC, N), lambda e: (e, 0)),
            pl.BlockSpec((N, DH), lambda e: (0, 0)),
            pl.BlockSpec((N, DY), lambda e: (0, 0)),
        ],
        out_specs=[
            pl.BlockSpec((B, 1), lambda e: (0, 0)),
            pl.BlockSpec((1, 1), lambda e: (0, 0)),
            pl.BlockSpec((1, 1), lambda e: (0, 0)),
            pl.BlockSpec((B, DH), lambda e: (0, 0)),
            pl.BlockSpec((B, DY), lambda e: (0, 0)),
            pl.BlockSpec((B, DI), lambda e: (0, 0)),
        ],
        out_shape=[jax.ShapeDtypeStruct((B, 1), jnp.int32),
                   jax.ShapeDtypeStruct((1, 1), jnp.float32),
                   jax.ShapeDtypeStruct((1, 1), jnp.float32),
                   jax.ShapeDtypeStruct((B, DH), jnp.float32),
                   jax.ShapeDtypeStruct((B, DY), jnp.float32),
                   jax.ShapeDtypeStruct((B, DI), jnp.float32)],
        scratch_shapes=[pltpu.VMEM((B, CH), jnp.float32),
                        pltpu.VMEM((B, N), jnp.float32),
                        pltpu.VMEM((8, EC), jnp.float32),
                        pltpu.VMEM((1, N), jnp.float32)],
        compiler_params=pltpu.CompilerParams(vmem_limit_bytes=63 * 1024 * 1024),
    )(x, Wf1, bf1.reshape(1, CH), Wx1, bx1.reshape(1, CH), Wx2, bx2.reshape(1, DI),
      Wf2.reshape(CH, NE, 1, EC), bf2.reshape(NE, 1, EC), embed, bd1, bd2)

    ind_flat = ind2.reshape(B)

    def _wspec(k, shape):
        return pl.BlockSpec(shape, lambda b, ind, k=k: (ind[SG * b + k],) + (0,) * (len(shape) - 1))

    dec = pl.pallas_call(
        _dec_body,
        grid_spec=pltpu.PrefetchScalarGridSpec(
            num_scalar_prefetch=1,
            grid=(B // SG,),
            in_specs=(
                [pl.BlockSpec((SG, DI), lambda b, ind: (b, 0)),
                 pl.BlockSpec((SG, DH), lambda b, ind: (b, 0)),
                 pl.BlockSpec((SG, DY), lambda b, ind: (b, 0))]
                + [_wspec(k, (1, DI, DH)) for k in range(SG)]
                + [_wspec(k, (1, DH, DY)) for k in range(SG)]
            ),
            out_specs=pl.BlockSpec((SG, DY), lambda b, ind: (b, 0)),
        ),
        out_shape=jax.ShapeDtypeStruct((B, DY), jnp.float32),
    )(ind_flat, ex, b1g, b2g, *([Wd1] * SG), *([Wd2] * SG))

    return (dec, diff2[0, 0], ind_flat, perp2[0, 0])
